# baseline (speedup 1.0000x reference)
import functools

import jax
import jax.numpy as jnp
from jax import lax
from jax.experimental import pallas as pl
from jax.experimental.pallas import tpu as pltpu
from jax.experimental.pallas import tpu_sc as plsc


def _count_body(vmax, d_ref, idx_ref):
    x = d_ref[...]
    cnt = jnp.sum(jnp.where(jnp.abs(x) == 1, 1, 0).astype(jnp.int32), axis=-1)
    idx_ref[...] = jnp.minimum(cnt, vmax).reshape(idx_ref.shape)


def _counts(distances, vmax):
    b, r, n = distances.shape
    rows = b * r
    block_rows = 256
    nblk = rows // block_rows
    d2 = distances.reshape(nblk, block_rows, n)
    idx3 = pl.pallas_call(
        functools.partial(_count_body, vmax),
        grid=(nblk,),
        in_specs=[pl.BlockSpec((1, block_rows, n), lambda i: (i, 0, 0))],
        out_specs=pl.BlockSpec((1, 1, block_rows), lambda i: (i, 0, 0)),
        out_shape=jax.ShapeDtypeStruct((nblk, 1, block_rows), jnp.int32),
    )(d2)
    return idx3.reshape(rows)


def _make_gather(rows, v, d):
    info = plsc.get_sparse_core_info()
    nw = info.num_cores * info.num_subcores
    b_per_w = rows // nw
    chunk = 64
    n_chunks = b_per_w // chunk
    mesh = plsc.VectorSubcoreMesh(core_axis_name="c", subcore_axis_name="s")

    @functools.partial(
        pl.kernel,
        out_type=jax.ShapeDtypeStruct((rows, d), jnp.float32),
        mesh=mesh,
        scratch_types=[
            pltpu.VMEM((chunk,), jnp.int32),
            pltpu.VMEM((chunk, d), jnp.float32),
            pltpu.SemaphoreType.DMA,
        ],
    )
    def gather(table_hbm, idx_hbm, out_hbm, idx_v, rows_v, sem):
        wid = lax.axis_index("s") * info.num_cores + lax.axis_index("c")
        base = wid * b_per_w

        def body(i, carry):
            off = base + i * chunk
            pltpu.sync_copy(idx_hbm.at[pl.ds(off, chunk)], idx_v)
            pltpu.async_copy(table_hbm.at[idx_v], rows_v, sem).wait()
            pltpu.sync_copy(rows_v, out_hbm.at[pl.ds(off, chunk)])
            return carry

        lax.fori_loop(0, n_chunks, body, 0)

    return gather


def kernel(distances, table):
    b, r, n = distances.shape
    v, d = table.shape
    rows = b * r
    idx = _counts(distances, v - 1)
    out = _make_gather(rows, v, d)(table, idx)
    return out.reshape(b, r, d)


# speedup vs baseline: 1.0574x; 1.0574x over previous
import functools

import jax
import jax.numpy as jnp
from jax import lax
from jax.experimental import pallas as pl
from jax.experimental.pallas import tpu as pltpu
from jax.experimental.pallas import tpu_sc as plsc


def _count_body(vmax, d_ref, idx_ref):
    x = d_ref[...]
    cnt = jnp.sum(jnp.where(jnp.abs(x) == 1, 1, 0).astype(jnp.int32), axis=-1)
    idx_ref[...] = jnp.minimum(cnt, vmax).reshape(idx_ref.shape)


def _counts(distances, vmax):
    b, r, n = distances.shape
    rows = b * r
    block_rows = 512
    nblk = rows // block_rows
    d2 = distances.reshape(nblk, block_rows, n)
    idx3 = pl.pallas_call(
        functools.partial(_count_body, vmax),
        grid=(nblk,),
        in_specs=[pl.BlockSpec((1, block_rows, n), lambda i: (i, 0, 0))],
        out_specs=pl.BlockSpec((1, 1, block_rows), lambda i: (i, 0, 0)),
        out_shape=jax.ShapeDtypeStruct((nblk, 1, block_rows), jnp.int32),
    )(d2)
    return idx3.reshape(rows)


_CHUNK = 64
_NBUF = 3


def _make_gather(rows, d):
    info = plsc.get_sparse_core_info()
    nw = info.num_cores * info.num_subcores
    b_per_w = rows // nw
    n_chunks = b_per_w // _CHUNK
    mesh = plsc.VectorSubcoreMesh(core_axis_name="c", subcore_axis_name="s")

    scratch = (
        [pltpu.VMEM((b_per_w,), jnp.int32)]
        + [pltpu.VMEM((_CHUNK, d), jnp.float32) for _ in range(_NBUF)]
        + [pltpu.SemaphoreType.DMA for _ in range(2 * _NBUF)]
    )

    @functools.partial(
        pl.kernel,
        out_type=jax.ShapeDtypeStruct((rows, d), jnp.float32),
        mesh=mesh,
        scratch_types=scratch,
    )
    def gather(table_hbm, idx_hbm, out_hbm, idx_v, *bufs_sems):
        bufs = bufs_sems[:_NBUF]
        gsem = bufs_sems[_NBUF:2 * _NBUF]
        ssem = bufs_sems[2 * _NBUF:]
        wid = lax.axis_index("s") * info.num_cores + lax.axis_index("c")
        base = wid * b_per_w

        pltpu.sync_copy(idx_hbm.at[pl.ds(base, b_per_w)], idx_v)

        gathers = [None] * _NBUF
        scatters = [None] * _NBUF

        def start_gather(c):
            s = c % _NBUF
            gathers[s] = pltpu.async_copy(
                table_hbm.at[idx_v.at[pl.ds(c * _CHUNK, _CHUNK)]], bufs[s], gsem[s])

        for c in range(min(_NBUF, n_chunks)):
            start_gather(c)
        for c in range(n_chunks):
            s = c % _NBUF
            gathers[s].wait()
            scatters[s] = pltpu.async_copy(
                bufs[s], out_hbm.at[pl.ds(base + c * _CHUNK, _CHUNK)], ssem[s])
            nxt = c + _NBUF
            if nxt < n_chunks:
                scatters[s].wait()
                scatters[s] = None
                start_gather(nxt)
        for s in range(_NBUF):
            if scatters[s] is not None:
                scatters[s].wait()

    return gather


def kernel(distances, table):
    b, r, n = distances.shape
    v, d = table.shape
    rows = b * r
    idx = _counts(distances, v - 1)
    out = _make_gather(rows, d)(table, idx)
    return out.reshape(b, r, d)


# speedup vs baseline: 1.8065x; 1.7085x over previous
import functools

import jax
import jax.numpy as jnp
from jax import lax
from jax.experimental import pallas as pl
from jax.experimental.pallas import tpu as pltpu
from jax.experimental.pallas import tpu_sc as plsc


_RB = 32
_L = 16


def _make_degree(rows, n, vmax):
    info = plsc.get_sparse_core_info()
    nw = info.num_cores * info.num_subcores
    b_per_w = rows // nw
    n_blocks = b_per_w // _RB
    mesh = plsc.VectorSubcoreMesh(core_axis_name="c", subcore_axis_name="s")

    scratch = [
        pltpu.VMEM((_RB, n), jnp.int32),
        pltpu.VMEM((_RB, n), jnp.int32),
        pltpu.VMEM((b_per_w,), jnp.int32),
        pltpu.SemaphoreType.DMA,
        pltpu.SemaphoreType.DMA,
    ]

    @functools.partial(
        pl.kernel,
        out_type=jax.ShapeDtypeStruct((rows,), jnp.int32),
        mesh=mesh,
        scratch_types=scratch,
        compiler_params=pltpu.CompilerParams(use_tc_tiling_on_sc=False,
                                             needs_layout_passes=False),
    )
    def degree(d_hbm, idx_hbm, d0, d1, cnt_v, sem0, sem1):
        wid = lax.axis_index("s") * info.num_cores + lax.axis_index("c")
        base = wid * b_per_w
        bufs = (d0, d1)
        sems = (sem0, sem1)

        def start(g):
            return pltpu.async_copy(
                d_hbm.at[pl.ds(base + g * _RB, _RB)], bufs[g % 2], sems[g % 2])

        copies = [None, None]
        copies[0] = start(0)
        if n_blocks > 1:
            copies[1] = start(1)

        lane_iota = lax.iota(jnp.int32, _L)

        for g in range(n_blocks):
            buf = bufs[g % 2]
            copies[g % 2].wait()

            def group_body(q, _, buf=buf, g=g):
                rows = q * _L + lane_iota

                def col_body(t, acc, buf=buf, rows=rows):
                    c0 = t * _L
                    for dgn in range(_L):
                        cols = c0 + ((lane_iota + dgn) & (_L - 1))
                        v = plsc.load_gather(buf, [rows, cols])
                        acc = acc + jnp.where(jnp.abs(v) == 1, 1, 0).astype(jnp.int32)
                    return acc

                acc = lax.fori_loop(0, n // _L, col_body,
                                    jnp.zeros((_L,), jnp.int32))
                cnt_v[pl.ds(g * _RB + q * _L, _L)] = jnp.minimum(acc, vmax)
                return _

            lax.fori_loop(0, _RB // _L, group_body, 0)
            if g + 2 < n_blocks:
                copies[g % 2] = start(g + 2)

        pltpu.sync_copy(cnt_v, idx_hbm.at[pl.ds(base, b_per_w)])

    return degree


_BR = 512


def _lookup_body(v, idx_ref, table_ref, out_ref):
    br = out_ref.shape[0]
    ids = idx_ref[...].reshape(br, 1)
    iot = lax.broadcasted_iota(jnp.int32, (br, v), 1)
    onehot = (iot == ids).astype(jnp.float32)
    out_ref[...] = jnp.dot(onehot, table_ref[...],
                           preferred_element_type=jnp.float32)


def _lookup(idx, table):
    rows = idx.shape[0]
    v, d = table.shape
    nblk = rows // _BR
    idx3 = idx.reshape(nblk, 1, _BR)
    return pl.pallas_call(
        functools.partial(_lookup_body, v),
        grid=(nblk,),
        in_specs=[
            pl.BlockSpec((1, 1, _BR), lambda i: (i, 0, 0)),
            pl.BlockSpec((v, d), lambda i: (0, 0)),
        ],
        out_specs=pl.BlockSpec((_BR, d), lambda i: (i, 0)),
        out_shape=jax.ShapeDtypeStruct((rows, d), jnp.float32),
    )(idx3, table)


def kernel(distances, table):
    b, r, n = distances.shape
    v, d = table.shape
    rows = b * r
    d2 = distances.reshape(rows, n)
    idx = _make_degree(rows, n, v - 1)(d2)
    out = _lookup(idx, table)
    return out.reshape(b, r, d)


# speedup vs baseline: 2.3810x; 1.3180x over previous
import functools

import jax
import jax.numpy as jnp
from jax import lax
from jax.experimental import pallas as pl
from jax.experimental.pallas import tpu as pltpu
from jax.experimental.pallas import tpu_sc as plsc


_RB = 32
_L = 16


def _make_degree(rows, n, vmax):
    info = plsc.get_sparse_core_info()
    nw = info.num_cores * info.num_subcores
    b_per_w = rows // nw
    n_blocks = b_per_w // _RB
    mesh = plsc.VectorSubcoreMesh(core_axis_name="c", subcore_axis_name="s")

    scratch = [
        pltpu.VMEM((_RB, n), jnp.int32),
        pltpu.VMEM((_RB, n), jnp.int32),
        pltpu.VMEM((b_per_w,), jnp.int32),
        pltpu.SemaphoreType.DMA,
        pltpu.SemaphoreType.DMA,
    ]

    @functools.partial(
        pl.kernel,
        out_type=jax.ShapeDtypeStruct((rows,), jnp.int32),
        mesh=mesh,
        scratch_types=scratch,
        compiler_params=pltpu.CompilerParams(needs_layout_passes=False),
    )
    def degree(d_hbm, idx_hbm, d0, d1, cnt_v, sem0, sem1):
        wid = lax.axis_index("s") * info.num_cores + lax.axis_index("c")
        base = wid * b_per_w
        bufs = (d0, d1)
        sems = (sem0, sem1)

        def start(g):
            return pltpu.async_copy(
                d_hbm.at[pl.ds(base + g * _RB, _RB)], bufs[g % 2], sems[g % 2])

        copies = [None, None]
        copies[0] = start(0)
        if n_blocks > 1:
            copies[1] = start(1)

        lane_iota = lax.iota(jnp.int32, _L)

        for g in range(n_blocks):
            buf = bufs[g % 2]
            copies[g % 2].wait()

            def group_body(q, _, buf=buf, g=g):
                rows = q * _L + lane_iota

                def col_body(t, acc, buf=buf, rows=rows):
                    c0 = t * _L
                    for dgn in range(_L):
                        cols = c0 + ((lane_iota + dgn) & (_L - 1))
                        v = plsc.load_gather(buf, [rows, cols])
                        acc = acc + jnp.where(jnp.abs(v) == 1, 1, 0).astype(jnp.int32)
                    return acc

                acc = lax.fori_loop(0, n // _L, col_body,
                                    jnp.zeros((_L,), jnp.int32))
                cnt_v[pl.ds(g * _RB + q * _L, _L)] = jnp.minimum(acc, vmax)
                return _

            lax.fori_loop(0, _RB // _L, group_body, 0)
            if g + 2 < n_blocks:
                copies[g % 2] = start(g + 2)

        pltpu.sync_copy(cnt_v, idx_hbm.at[pl.ds(base, b_per_w)])

    return degree


_BR = 512


def _lookup_body(v, idx_ref, table_ref, out_ref):
    br = out_ref.shape[0]
    ids = idx_ref[...].reshape(br, 1)
    iot = lax.broadcasted_iota(jnp.int32, (br, v), 1)
    onehot = (iot == ids).astype(jnp.float32)
    out_ref[...] = jnp.dot(onehot, table_ref[...],
                           preferred_element_type=jnp.float32)


def _lookup(idx, table):
    rows = idx.shape[0]
    v, d = table.shape
    nblk = rows // _BR
    idx3 = idx.reshape(nblk, 1, _BR)
    return pl.pallas_call(
        functools.partial(_lookup_body, v),
        grid=(nblk,),
        in_specs=[
            pl.BlockSpec((1, 1, _BR), lambda i: (i, 0, 0)),
            pl.BlockSpec((v, d), lambda i: (0, 0)),
        ],
        out_specs=pl.BlockSpec((_BR, d), lambda i: (i, 0)),
        out_shape=jax.ShapeDtypeStruct((rows, d), jnp.float32),
    )(idx3, table)


def kernel(distances, table):
    b, r, n = distances.shape
    v, d = table.shape
    rows = b * r
    d2 = distances.reshape(rows, n)
    idx = _make_degree(rows, n, v - 1)(d2)
    out = _lookup(idx, table)
    return out.reshape(b, r, d)


# speedup vs baseline: 2.5913x; 1.0883x over previous
import functools

import jax
import jax.numpy as jnp
from jax import lax
from jax.experimental import pallas as pl
from jax.experimental.pallas import tpu as pltpu
from jax.experimental.pallas import tpu_sc as plsc


_RB = 32
_L = 16


def _make_degree(rows, n, vmax):
    info = plsc.get_sparse_core_info()
    nw = info.num_cores * info.num_subcores
    b_per_w = rows // nw
    n_blocks = b_per_w // _RB
    mesh = plsc.VectorSubcoreMesh(core_axis_name="c", subcore_axis_name="s")

    scratch = [
        pltpu.VMEM((_RB, n), jnp.int32),
        pltpu.VMEM((_RB, n), jnp.int32),
        pltpu.VMEM((b_per_w,), jnp.int32),
        pltpu.SemaphoreType.DMA,
        pltpu.SemaphoreType.DMA,
    ]

    @functools.partial(
        pl.kernel,
        out_type=jax.ShapeDtypeStruct((rows,), jnp.int32),
        mesh=mesh,
        scratch_types=scratch,
        compiler_params=pltpu.CompilerParams(needs_layout_passes=False),
    )
    def degree(d_hbm, idx_hbm, d0, d1, cnt_v, sem0, sem1):
        wid = lax.axis_index("s") * info.num_cores + lax.axis_index("c")
        base = wid * b_per_w
        bufs = (d0, d1)
        sems = (sem0, sem1)

        def start(g):
            return pltpu.async_copy(
                d_hbm.at[pl.ds(base + g * _RB, _RB)], bufs[g % 2], sems[g % 2])

        copies = [None, None]
        copies[0] = start(0)
        if n_blocks > 1:
            copies[1] = start(1)

        lane_iota = lax.iota(jnp.int32, _L)

        for g in range(n_blocks):
            buf = bufs[g % 2]
            copies[g % 2].wait()

            def group_body(q, _, buf=buf, g=g):
                rows = q * _L + lane_iota

                def col_body(t, acc, buf=buf, rows=rows):
                    c0 = t * _L
                    for dgn in range(_L):
                        cols = c0 + ((lane_iota + dgn) & (_L - 1))
                        v = plsc.load_gather(buf, [rows, cols])
                        acc = acc + jnp.where(jnp.abs(v) == 1, 1, 0).astype(jnp.int32)
                    return acc

                acc = lax.fori_loop(0, n // _L, col_body,
                                    jnp.zeros((_L,), jnp.int32))
                cnt_v[pl.ds(g * _RB + q * _L, _L)] = jnp.minimum(acc, vmax)
                return _

            lax.fori_loop(0, _RB // _L, group_body, 0)
            if g + 2 < n_blocks:
                copies[g % 2] = start(g + 2)

        pltpu.sync_copy(cnt_v, idx_hbm.at[pl.ds(base, b_per_w)])

    return degree


_BR = 2048


def _lookup_body(v, idx_ref, table_ref, out_ref):
    br = out_ref.shape[0]
    ids = idx_ref[...].reshape(br, 1)
    iot = lax.broadcasted_iota(jnp.int32, (br, v), 1)
    onehot = (iot == ids).astype(jnp.float32)
    out_ref[...] = jnp.dot(onehot, table_ref[...],
                           preferred_element_type=jnp.float32)


def _lookup(idx, table):
    rows = idx.shape[0]
    v, d = table.shape
    nblk = rows // _BR
    idx3 = idx.reshape(nblk, 1, _BR)
    return pl.pallas_call(
        functools.partial(_lookup_body, v),
        grid=(nblk,),
        in_specs=[
            pl.BlockSpec((1, 1, _BR), lambda i: (i, 0, 0)),
            pl.BlockSpec((v, d), lambda i: (0, 0)),
        ],
        out_specs=pl.BlockSpec((_BR, d), lambda i: (i, 0)),
        out_shape=jax.ShapeDtypeStruct((rows, d), jnp.float32),
    )(idx3, table)


def kernel(distances, table):
    b, r, n = distances.shape
    v, d = table.shape
    rows = b * r
    d2 = distances.reshape(rows, n)
    idx = _make_degree(rows, n, v - 1)(d2)
    out = _lookup(idx, table)
    return out.reshape(b, r, d)
